# baseline (speedup 1.0000x reference)
import functools

import jax
import jax.numpy as jnp
from jax import lax
from jax.experimental import pallas as pl
from jax.experimental.pallas import tpu as pltpu
from jax.experimental.pallas import tpu_sc as plsc

_B, _S, _D = 4, 4096, 1024
_NC, _NS = 2, 16
_NW = _NC * _NS
_ROWS_W = _S // _NW
_CH = 16
_NCHUNK = _ROWS_W // _CH
_NVEC = _CH * _D // 16
_CPR = _D // 16
_NSLOT = 5
_LOOKAHEAD = 2


def _sc_add(x_hbm, pos_hbm, out_hbm, xbuf, pbuf,
            lsem0, lsem1, lsem2, lsem3, lsem4,
            ssem0, ssem1, ssem2, ssem3, ssem4, psem0, psem1):
    wid = lax.axis_index("s") * _NC + lax.axis_index("c")
    row0 = wid * _ROWS_W
    lsems = (lsem0, lsem1, lsem2, lsem3, lsem4)
    ssems = (ssem0, ssem1, ssem2, ssem3, ssem4)
    psems = (psem0, psem1)

    def load_x(i):
        c, b = divmod(i, _B)
        return pltpu.async_copy(
            x_hbm.at[b, pl.ds(row0 + c * _CH, _CH)], xbuf.at[i % _NSLOT],
            lsems[i % _NSLOT])

    def load_pos(c):
        return pltpu.async_copy(
            pos_hbm.at[pl.ds(row0 + c * _CH, _CH)], pbuf.at[c % 2],
            psems[c % 2])

    n_items = _NCHUNK * _B
    load_h = [None] * n_items
    store_h = [None] * n_items
    pos_h = [None] * _NCHUNK

    pos_h[0] = load_pos(0)
    for i in range(_LOOKAHEAD):
        load_h[i] = load_x(i)

    for i in range(n_items):
        c, b = divmod(i, _B)
        if b == 0 and c + 1 < _NCHUNK:
            pos_h[c + 1] = load_pos(c + 1)
        j = i + _LOOKAHEAD
        if j < n_items:
            if j >= _NSLOT:
                store_h[j - _NSLOT].wait()
            load_h[j] = load_x(j)
        if b == 0:
            pos_h[c].wait()
        load_h[i].wait()

        xb = xbuf.at[i % _NSLOT]
        pb = pbuf.at[c % 2]

        @plsc.parallel_loop(0, _NVEC, unroll=8)
        def add_body(j):
            r = j >> 6
            cc = (j & (_CPR - 1)) * 16
            xb[r, pl.ds(cc, 16)] = xb[r, pl.ds(cc, 16)] + pb[r, pl.ds(cc, 16)]

        store_h[i] = pltpu.async_copy(
            xbuf.at[i % _NSLOT], out_hbm.at[b, pl.ds(row0 + c * _CH, _CH)],
            ssems[i % _NSLOT])

    for i in range(n_items - _NSLOT, n_items):
        store_h[i].wait()


_mesh = plsc.VectorSubcoreMesh(core_axis_name="c", subcore_axis_name="s")

_call = functools.partial(
    pl.kernel,
    out_type=jax.ShapeDtypeStruct((_B, _S, _D), jnp.float32),
    mesh=_mesh,
    scratch_types=[
        pltpu.VMEM((_NSLOT, _CH, _D), jnp.float32),
        pltpu.VMEM((2, _CH, _D), jnp.float32),
        pltpu.SemaphoreType.DMA,
        pltpu.SemaphoreType.DMA,
        pltpu.SemaphoreType.DMA,
        pltpu.SemaphoreType.DMA,
        pltpu.SemaphoreType.DMA,
        pltpu.SemaphoreType.DMA,
        pltpu.SemaphoreType.DMA,
        pltpu.SemaphoreType.DMA,
        pltpu.SemaphoreType.DMA,
        pltpu.SemaphoreType.DMA,
        pltpu.SemaphoreType.DMA,
        pltpu.SemaphoreType.DMA,
    ],
)(_sc_add)


@jax.jit
def kernel(x, pos_table):
    return _call(x, pos_table)


# speedup vs baseline: 1.0158x; 1.0158x over previous
import functools

import jax
import jax.numpy as jnp
from jax import lax
from jax.experimental import pallas as pl
from jax.experimental.pallas import tpu as pltpu
from jax.experimental.pallas import tpu_sc as plsc

_B, _S, _D = 4, 4096, 1024
_NC, _NS = 2, 16
_NW = _NC * _NS
_ROWS_W = _S // _NW
_CH = 16
_NCHUNK = _ROWS_W // _CH
_NVEC = _CH * _D // 16
_CPR = _D // 16
_NSLOT = 5
_LOOKAHEAD = 3


def _sc_add(x_hbm, pos_hbm, out_hbm, xbuf, pbuf,
            lsem0, lsem1, lsem2, lsem3, lsem4,
            ssem0, ssem1, ssem2, ssem3, ssem4, psem0, psem1):
    wid = lax.axis_index("s") * _NC + lax.axis_index("c")
    row0 = wid * _ROWS_W
    lsems = (lsem0, lsem1, lsem2, lsem3, lsem4)
    ssems = (ssem0, ssem1, ssem2, ssem3, ssem4)
    psems = (psem0, psem1)

    def load_x(i):
        c, b = divmod(i, _B)
        return pltpu.async_copy(
            x_hbm.at[b, pl.ds(row0 + c * _CH, _CH)], xbuf.at[i % _NSLOT],
            lsems[i % _NSLOT])

    def load_pos(c):
        return pltpu.async_copy(
            pos_hbm.at[pl.ds(row0 + c * _CH, _CH)], pbuf.at[c % 2],
            psems[c % 2])

    n_items = _NCHUNK * _B
    load_h = [None] * n_items
    store_h = [None] * n_items
    pos_h = [None] * _NCHUNK

    pos_h[0] = load_pos(0)
    for i in range(_LOOKAHEAD):
        load_h[i] = load_x(i)

    for i in range(n_items):
        c, b = divmod(i, _B)
        if b == 0 and c + 1 < _NCHUNK:
            pos_h[c + 1] = load_pos(c + 1)
        j = i + _LOOKAHEAD
        if j < n_items:
            if j >= _NSLOT:
                store_h[j - _NSLOT].wait()
            load_h[j] = load_x(j)
        if b == 0:
            pos_h[c].wait()
        load_h[i].wait()

        xb = xbuf.at[i % _NSLOT]
        pb = pbuf.at[c % 2]

        @plsc.parallel_loop(0, _NVEC, unroll=8)
        def add_body(j):
            r = j >> 6
            cc = (j & (_CPR - 1)) * 16
            xb[r, pl.ds(cc, 16)] = xb[r, pl.ds(cc, 16)] + pb[r, pl.ds(cc, 16)]

        store_h[i] = pltpu.async_copy(
            xbuf.at[i % _NSLOT], out_hbm.at[b, pl.ds(row0 + c * _CH, _CH)],
            ssems[i % _NSLOT])

    for i in range(n_items - _NSLOT, n_items):
        store_h[i].wait()


_mesh = plsc.VectorSubcoreMesh(core_axis_name="c", subcore_axis_name="s")

_call = functools.partial(
    pl.kernel,
    out_type=jax.ShapeDtypeStruct((_B, _S, _D), jnp.float32),
    mesh=_mesh,
    scratch_types=[
        pltpu.VMEM((_NSLOT, _CH, _D), jnp.float32),
        pltpu.VMEM((2, _CH, _D), jnp.float32),
        pltpu.SemaphoreType.DMA,
        pltpu.SemaphoreType.DMA,
        pltpu.SemaphoreType.DMA,
        pltpu.SemaphoreType.DMA,
        pltpu.SemaphoreType.DMA,
        pltpu.SemaphoreType.DMA,
        pltpu.SemaphoreType.DMA,
        pltpu.SemaphoreType.DMA,
        pltpu.SemaphoreType.DMA,
        pltpu.SemaphoreType.DMA,
        pltpu.SemaphoreType.DMA,
        pltpu.SemaphoreType.DMA,
    ],
)(_sc_add)


@jax.jit
def kernel(x, pos_table):
    return _call(x, pos_table)


# speedup vs baseline: 1.0180x; 1.0022x over previous
import functools

import jax
import jax.numpy as jnp
from jax import lax
from jax.experimental import pallas as pl
from jax.experimental.pallas import tpu as pltpu
from jax.experimental.pallas import tpu_sc as plsc

_B, _S, _D = 4, 4096, 1024
_NC, _NS = 2, 16
_NW = _NC * _NS
_ROWS_W = _S // _NW
_CH = 16
_NCHUNK = _ROWS_W // _CH
_NVEC = _CH * _D // 16
_CPR = _D // 16
_NSLOT = 5
_LOOKAHEAD = 3


def _sc_add(x_hbm, pos_hbm, out_hbm, xbuf, pbuf,
            lsem0, lsem1, lsem2, lsem3, lsem4,
            ssem0, ssem1, ssem2, ssem3, ssem4, psem0, psem1):
    wid = lax.axis_index("s") * _NC + lax.axis_index("c")
    row0 = wid * _ROWS_W
    lsems = (lsem0, lsem1, lsem2, lsem3, lsem4)
    ssems = (ssem0, ssem1, ssem2, ssem3, ssem4)
    psems = (psem0, psem1)

    def load_x(i):
        c, b = divmod(i, _B)
        return pltpu.async_copy(
            x_hbm.at[b, pl.ds(row0 + c * _CH, _CH)], xbuf.at[i % _NSLOT],
            lsems[i % _NSLOT])

    def load_pos(c):
        return pltpu.async_copy(
            pos_hbm.at[pl.ds(row0 + c * _CH, _CH)], pbuf.at[c % 2],
            psems[c % 2])

    n_items = _NCHUNK * _B
    load_h = [None] * n_items
    store_h = [None] * n_items
    pos_h = [None] * _NCHUNK

    pos_h[0] = load_pos(0)
    for i in range(_LOOKAHEAD):
        load_h[i] = load_x(i)

    for i in range(n_items):
        c, b = divmod(i, _B)
        if b == 0 and c + 1 < _NCHUNK:
            pos_h[c + 1] = load_pos(c + 1)
        j = i + _LOOKAHEAD
        if j < n_items:
            if j >= _NSLOT:
                store_h[j - _NSLOT].wait()
            load_h[j] = load_x(j)
        if b == 0:
            pos_h[c].wait()
        load_h[i].wait()

        xb = xbuf.at[i % _NSLOT]
        pb = pbuf.at[c % 2]

        @plsc.parallel_loop(0, _NVEC, unroll=8)
        def add_body(j):
            r = j >> 6
            cc = (j & (_CPR - 1)) * 16
            plsc.addupdate(xb.at[r, pl.ds(cc, 16)], pb[r, pl.ds(cc, 16)])

        store_h[i] = pltpu.async_copy(
            xbuf.at[i % _NSLOT], out_hbm.at[b, pl.ds(row0 + c * _CH, _CH)],
            ssems[i % _NSLOT])

    for i in range(n_items - _NSLOT, n_items):
        store_h[i].wait()


_mesh = plsc.VectorSubcoreMesh(core_axis_name="c", subcore_axis_name="s")

_call = functools.partial(
    pl.kernel,
    out_type=jax.ShapeDtypeStruct((_B, _S, _D), jnp.float32),
    mesh=_mesh,
    scratch_types=[
        pltpu.VMEM((_NSLOT, _CH, _D), jnp.float32),
        pltpu.VMEM((2, _CH, _D), jnp.float32),
        pltpu.SemaphoreType.DMA,
        pltpu.SemaphoreType.DMA,
        pltpu.SemaphoreType.DMA,
        pltpu.SemaphoreType.DMA,
        pltpu.SemaphoreType.DMA,
        pltpu.SemaphoreType.DMA,
        pltpu.SemaphoreType.DMA,
        pltpu.SemaphoreType.DMA,
        pltpu.SemaphoreType.DMA,
        pltpu.SemaphoreType.DMA,
        pltpu.SemaphoreType.DMA,
        pltpu.SemaphoreType.DMA,
    ],
)(_sc_add)


@jax.jit
def kernel(x, pos_table):
    return _call(x, pos_table)
